# baseline (speedup 1.0000x reference)
import jax
import jax.numpy as jnp
from jax.experimental import pallas as pl
from jax.experimental.pallas import tpu as pltpu

_EMBED_DIM = 64
_N_EMBED = 1024
_BLOCK_M = 1024
_RCH = 128


def _vq_block(x_ref, xp_ref, x2_ref, emb_ref, emb2t_ref,
              idx_ref, quant_ref, codes_ref,
              e2b_ref, idx_s_ref):
    @pl.when(pl.program_id(0) == 0)
    def _():
        emb0 = emb_ref[...]
        e2row = jnp.sum(emb0 * emb0, axis=0, keepdims=True)
        e2b_ref[...] = jnp.broadcast_to(e2row.T, (_N_EMBED, _BLOCK_M))

    idxp = idx_s_ref[0, :]
    fiota = jax.lax.broadcasted_iota(jnp.int32, (_N_EMBED, _BLOCK_M), 0)
    onehot_t = (fiota == idxp[None, :]).astype(jnp.float32)
    emb = emb_ref[...]
    qt = jnp.dot(emb, onehot_t, preferred_element_type=jnp.float32)
    q = qt.T
    xp = xp_ref[...]
    idx_ref[0, 0, :] = idxp
    quant_ref[...] = xp + (q - xp)
    codes_ref[...] = jnp.concatenate([xp, q], axis=1)

    x = x_ref[...]
    emb2t = emb2t_ref[...]
    x2 = x2_ref[0, 0, :][None, :]
    xt = x.T
    dot2t = jnp.dot(emb2t, xt, preferred_element_type=jnp.float32)

    nch = _N_EMBED // _RCH
    e2b = e2b_ref[...]
    pairs = [
        ((x2 - dot2t[k * _RCH:(k + 1) * _RCH, :]) + e2b[k * _RCH:(k + 1) * _RCH, :],
         jnp.full((_RCH, _BLOCK_M), k, jnp.int32))
        for k in range(nch)
    ]
    while len(pairs) > 1:
        nxt = []
        for a, b in zip(pairs[0::2], pairs[1::2]):
            lt = b[0] < a[0]
            nxt.append((jnp.where(lt, b[0], a[0]), jnp.where(lt, b[1], a[1])))
        pairs = nxt
    cm, jc = pairs[0]
    jiota = jax.lax.broadcasted_iota(jnp.int32, (_RCH, _BLOCK_M), 0)
    js = jc * _RCH + jiota
    mt = jnp.min(cm, axis=0, keepdims=True)
    idx = jnp.min(jnp.where(cm == mt, js, _N_EMBED), axis=0)
    idx_s_ref[0, :] = idx


def kernel(inputs, embedding):
    lead_shape = inputs.shape[:-1]
    flat = inputs.reshape(-1, _EMBED_DIM)
    n_rows = flat.shape[0]
    grid = n_rows // _BLOCK_M
    embt = embedding.T
    emb2t = embt + embt
    x2 = jnp.sum(flat * flat, axis=1).reshape(grid, 1, _BLOCK_M)

    last = grid - 1
    idx3, quant, codes = pl.pallas_call(
        _vq_block,
        grid=(grid + 1,),
        in_specs=[
            pl.BlockSpec((_BLOCK_M, _EMBED_DIM),
                         lambda i: (jnp.minimum(i, last), 0)),
            pl.BlockSpec((_BLOCK_M, _EMBED_DIM),
                         lambda i: (jnp.maximum(i - 1, 0), 0)),
            pl.BlockSpec((1, 1, _BLOCK_M),
                         lambda i: (jnp.minimum(i, last), 0, 0)),
            pl.BlockSpec((_EMBED_DIM, _N_EMBED), lambda i: (0, 0)),
            pl.BlockSpec((_N_EMBED, _EMBED_DIM), lambda i: (0, 0)),
        ],
        out_specs=[
            pl.BlockSpec((1, 1, _BLOCK_M),
                         lambda i: (jnp.maximum(i - 1, 0), 0, 0)),
            pl.BlockSpec((_BLOCK_M, _EMBED_DIM),
                         lambda i: (jnp.maximum(i - 1, 0), 0)),
            pl.BlockSpec((_BLOCK_M, 2 * _EMBED_DIM),
                         lambda i: (jnp.maximum(i - 1, 0), 0)),
        ],
        out_shape=[
            jax.ShapeDtypeStruct((grid, 1, _BLOCK_M), jnp.int32),
            jax.ShapeDtypeStruct((n_rows, _EMBED_DIM), jnp.float32),
            jax.ShapeDtypeStruct((n_rows, 2 * _EMBED_DIM), jnp.float32),
        ],
        scratch_shapes=[
            pltpu.VMEM((_N_EMBED, _BLOCK_M), jnp.float32),
            pltpu.VMEM((1, _BLOCK_M), jnp.int32),
        ],
    )(flat, flat, x2, embedding, emb2t)

    quantized = quant.reshape(inputs.shape)
    codes_out = codes.reshape(lead_shape + (2 * _EMBED_DIM,))
    encoding_indices = idx3.reshape(lead_shape)
    return (quantized, codes_out, encoding_indices)


# speedup vs baseline: 1.0186x; 1.0186x over previous
import jax
import jax.numpy as jnp
from jax.experimental import pallas as pl
from jax.experimental.pallas import tpu as pltpu

_EMBED_DIM = 64
_N_EMBED = 1024
_BLOCK_M = 1024
_RCH = 128


def _vq_block(x_ref, x2_ref, emb_ref, emb2t_ref,
              idx_ref, quant_ref, codes_ref,
              e2b_ref, idx_s_ref, x_s_ref):
    @pl.when(pl.program_id(0) == 0)
    def _():
        emb0 = emb_ref[...]
        e2row = jnp.sum(emb0 * emb0, axis=0, keepdims=True)
        e2b_ref[...] = jnp.broadcast_to(e2row.T, (_N_EMBED, _BLOCK_M))

    idxp = idx_s_ref[0, :]
    fiota = jax.lax.broadcasted_iota(jnp.int32, (_N_EMBED, _BLOCK_M), 0)
    onehot_t = (fiota == idxp[None, :]).astype(jnp.float32)
    emb = emb_ref[...]
    qt = jnp.dot(emb, onehot_t, preferred_element_type=jnp.float32)
    q = qt.T
    xp = x_s_ref[...]
    idx_ref[0, 0, :] = idxp
    quant_ref[...] = xp + (q - xp)
    codes_ref[...] = jnp.concatenate([xp, q], axis=1)

    x = x_ref[...]
    emb2t = emb2t_ref[...]
    x2 = x2_ref[0, 0, :][None, :]
    xt = x.T
    dot2t = jnp.dot(emb2t, xt, preferred_element_type=jnp.float32)

    nch = _N_EMBED // _RCH
    e2b = e2b_ref[...]
    pairs = [
        ((x2 - dot2t[k * _RCH:(k + 1) * _RCH, :]) + e2b[k * _RCH:(k + 1) * _RCH, :],
         jnp.full((_RCH, _BLOCK_M), k, jnp.int32))
        for k in range(nch)
    ]
    while len(pairs) > 1:
        nxt = []
        for a, b in zip(pairs[0::2], pairs[1::2]):
            lt = b[0] < a[0]
            nxt.append((jnp.where(lt, b[0], a[0]), jnp.where(lt, b[1], a[1])))
        pairs = nxt
    cm, jc = pairs[0]
    jiota = jax.lax.broadcasted_iota(jnp.int32, (_RCH, _BLOCK_M), 0)
    js = jc * _RCH + jiota
    mt = jnp.min(cm, axis=0, keepdims=True)
    idx = jnp.min(jnp.where(cm == mt, js, _N_EMBED), axis=0)
    idx_s_ref[0, :] = idx
    x_s_ref[...] = x


def kernel(inputs, embedding):
    lead_shape = inputs.shape[:-1]
    flat = inputs.reshape(-1, _EMBED_DIM)
    n_rows = flat.shape[0]
    grid = n_rows // _BLOCK_M
    embt = embedding.T
    emb2t = embt + embt
    x2 = jnp.sum(flat * flat, axis=1).reshape(grid, 1, _BLOCK_M)

    last = grid - 1
    idx3, quant, codes = pl.pallas_call(
        _vq_block,
        grid=(grid + 1,),
        in_specs=[
            pl.BlockSpec((_BLOCK_M, _EMBED_DIM),
                         lambda i: (jnp.minimum(i, last), 0)),
            pl.BlockSpec((1, 1, _BLOCK_M),
                         lambda i: (jnp.minimum(i, last), 0, 0)),
            pl.BlockSpec((_EMBED_DIM, _N_EMBED), lambda i: (0, 0)),
            pl.BlockSpec((_N_EMBED, _EMBED_DIM), lambda i: (0, 0)),
        ],
        out_specs=[
            pl.BlockSpec((1, 1, _BLOCK_M),
                         lambda i: (jnp.maximum(i - 1, 0), 0, 0)),
            pl.BlockSpec((_BLOCK_M, _EMBED_DIM),
                         lambda i: (jnp.maximum(i - 1, 0), 0)),
            pl.BlockSpec((_BLOCK_M, 2 * _EMBED_DIM),
                         lambda i: (jnp.maximum(i - 1, 0), 0)),
        ],
        out_shape=[
            jax.ShapeDtypeStruct((grid, 1, _BLOCK_M), jnp.int32),
            jax.ShapeDtypeStruct((n_rows, _EMBED_DIM), jnp.float32),
            jax.ShapeDtypeStruct((n_rows, 2 * _EMBED_DIM), jnp.float32),
        ],
        scratch_shapes=[
            pltpu.VMEM((_N_EMBED, _BLOCK_M), jnp.float32),
            pltpu.VMEM((1, _BLOCK_M), jnp.int32),
            pltpu.VMEM((_BLOCK_M, _EMBED_DIM), jnp.float32),
        ],
    )(flat, x2, embedding, emb2t)

    quantized = quant.reshape(inputs.shape)
    codes_out = codes.reshape(lead_shape + (2 * _EMBED_DIM,))
    encoding_indices = idx3.reshape(lead_shape)
    return (quantized, codes_out, encoding_indices)
